# baseline (speedup 1.0000x reference)
import jax
import jax.numpy as jnp
from jax import lax
from jax.experimental import pallas as pl
from jax.experimental.pallas import tpu as pltpu
from jax.experimental.pallas import tpu_sc as plsc

EMB_DIM = 128
HID_DIM = 256
BATCH = 16384

NUM_CORES = 2
NUM_SUBCORES = 16
NUM_WORKERS = NUM_CORES * NUM_SUBCORES

NCHUNK = 4
CHUNK = BATCH // NCHUNK
B_PER_W = CHUNK // NUM_WORKERS

BB = 2048
BLOCKS_PER_CHUNK = CHUNK // BB


def _make_gather_body(chunk_id):
    def body(idx_hbm, table_hbm, out_hbm, idx_v, rows_v, sem):
        wid = lax.axis_index("s") * NUM_CORES + lax.axis_index("c")
        src = chunk_id * CHUNK + wid * B_PER_W
        dst = wid * B_PER_W
        pltpu.sync_copy(idx_hbm.at[pl.ds(src, B_PER_W)], idx_v)
        pltpu.async_copy(table_hbm.at[idx_v], rows_v, sem).wait()
        pltpu.sync_copy(rows_v, out_hbm.at[pl.ds(dst, B_PER_W)])
    return body


def _make_gather(chunk_id):
    return pl.kernel(
        _make_gather_body(chunk_id),
        mesh=plsc.VectorSubcoreMesh(core_axis_name="c", subcore_axis_name="s"),
        out_type=jax.ShapeDtypeStruct((CHUNK, EMB_DIM), jnp.float32),
        scratch_types=[
            pltpu.VMEM((B_PER_W,), jnp.int32),
            pltpu.VMEM((B_PER_W, EMB_DIM), jnp.float32),
            pltpu.SemaphoreType.DMA,
        ],
    )


def _fc_ln(x, w, b, g, be):
    h = jnp.dot(x, w, preferred_element_type=jnp.float32)
    h = jnp.maximum(h + b, 0.0)
    mean = jnp.mean(h, axis=-1, keepdims=True)
    mean_sq = jnp.mean(jnp.square(h), axis=-1, keepdims=True)
    var = mean_sq - jnp.square(mean)
    r = lax.rsqrt(var + 1e-5)
    return (h - mean) * (r * g) + be


def _make_fc_body(row0, has_buf):
    def body(*refs):
        n = BLOCKS_PER_CHUNK
        if has_buf:
            x_hbm, w_ref, b_ref, g_ref, be_ref, buf_ref, o_hbm = refs[:7]
            rest = refs[7:]
            del buf_ref
        else:
            x_hbm, w_ref, b_ref, g_ref, be_ref, o_hbm = refs[:6]
            rest = refs[6:]
        xb = rest[:n]
        ob = rest[n:2 * n]
        isem, osem = rest[2 * n], rest[2 * n + 1]
        loads = []
        for i in range(n):
            loads.append(pltpu.async_copy(
                x_hbm.at[pl.ds(i * BB, BB), :], xb[i], isem.at[i]))
        stores = []
        for i in range(n):
            loads[i].wait()
            ob[i][...] = _fc_ln(xb[i][...], w_ref[...], b_ref[...],
                                g_ref[...], be_ref[...])
            stores.append(pltpu.async_copy(
                ob[i], o_hbm.at[pl.ds(row0 + i * BB, BB), :],
                osem.at[i]))
        for s in stores:
            s.wait()
    return body


def _fc_call(x, W, b2, g2, be2, row0, buf=None):
    has_buf = buf is not None
    any_spec = pl.BlockSpec(memory_space=pl.ANY)
    vmem_spec = pl.BlockSpec(memory_space=pltpu.VMEM)
    in_specs = [any_spec, vmem_spec, vmem_spec, vmem_spec, vmem_spec]
    args = [x, W, b2, g2, be2]
    kwargs = {}
    if has_buf:
        in_specs.append(any_spec)
        args.append(buf)
        kwargs["input_output_aliases"] = {5: 0}
    return pl.pallas_call(
        _make_fc_body(row0, has_buf),
        in_specs=in_specs,
        out_specs=any_spec,
        out_shape=jax.ShapeDtypeStruct((BATCH, HID_DIM), jnp.float32),
        scratch_shapes=(
            [pltpu.VMEM((BB, EMB_DIM), jnp.float32)] * BLOCKS_PER_CHUNK
            + [pltpu.VMEM((BB, HID_DIM), jnp.float32)] * BLOCKS_PER_CHUNK
            + [pltpu.SemaphoreType.DMA((BLOCKS_PER_CHUNK,)),
               pltpu.SemaphoreType.DMA((BLOCKS_PER_CHUNK,))]
        ),
        **kwargs,
    )(*args)


def kernel(item_input, table, W, b, gamma, beta):
    idx = item_input.astype(jnp.int32)
    b2 = b.reshape(1, HID_DIM)
    g2 = gamma.reshape(1, HID_DIM)
    be2 = beta.reshape(1, HID_DIM)

    xs = [_make_gather(c)(idx, table) for c in range(NCHUNK)]

    out = _fc_call(xs[0], W, b2, g2, be2, 0)
    for c in range(1, NCHUNK):
        out = _fc_call(xs[c], W, b2, g2, be2, c * CHUNK, buf=out)
    return out


# speedup vs baseline: 1.2424x; 1.2424x over previous
import jax
import jax.numpy as jnp
from jax import lax
from jax.experimental import pallas as pl
from jax.experimental.pallas import tpu as pltpu
from jax.experimental.pallas import tpu_sc as plsc

EMB_DIM = 128
HID_DIM = 256
BATCH = 16384

NUM_CORES = 2
NUM_SUBCORES = 16
NUM_WORKERS = NUM_CORES * NUM_SUBCORES

NCHUNK = 2
CHUNK = BATCH // NCHUNK
B_PER_W = CHUNK // NUM_WORKERS

BB = 2048
BLOCKS_PER_CHUNK = CHUNK // BB


def _make_gather_body(chunk_id):
    def body(idx_hbm, table_hbm, out_hbm, idx_v, rows_v, sem):
        wid = lax.axis_index("s") * NUM_CORES + lax.axis_index("c")
        src = chunk_id * CHUNK + wid * B_PER_W
        dst = wid * B_PER_W
        pltpu.sync_copy(idx_hbm.at[pl.ds(src, B_PER_W)], idx_v)
        pltpu.async_copy(table_hbm.at[idx_v], rows_v, sem).wait()
        pltpu.sync_copy(rows_v, out_hbm.at[pl.ds(dst, B_PER_W)])
    return body


def _fc_ln(x, w, b, g, be):
    h = jnp.dot(x, w, preferred_element_type=jnp.float32)
    h = jnp.maximum(h + b, 0.0)
    mean = jnp.mean(h, axis=-1, keepdims=True)
    mean_sq = jnp.mean(jnp.square(h), axis=-1, keepdims=True)
    var = mean_sq - jnp.square(mean)
    r = lax.rsqrt(var + 1e-5)
    return (h - mean) * (r * g) + be


def _fc_first_body(x_ref, w_ref, b_ref, g_ref, be_ref, o_ref):
    o_ref[...] = _fc_ln(x_ref[...], w_ref[...], b_ref[...], g_ref[...],
                        be_ref[...])


def _fc_second_body(x_ref, w_ref, b_ref, g_ref, be_ref, buf_ref, o_ref):
    del buf_ref
    o_ref[...] = _fc_ln(x_ref[...], w_ref[...], b_ref[...], g_ref[...],
                        be_ref[...])


def _make_gather(chunk_id):
    return pl.kernel(
        _make_gather_body(chunk_id),
        mesh=plsc.VectorSubcoreMesh(core_axis_name="c", subcore_axis_name="s"),
        out_type=jax.ShapeDtypeStruct((CHUNK, EMB_DIM), jnp.float32),
        scratch_types=[
            pltpu.VMEM((B_PER_W,), jnp.int32),
            pltpu.VMEM((B_PER_W, EMB_DIM), jnp.float32),
            pltpu.SemaphoreType.DMA,
        ],
    )


def kernel(item_input, table, W, b, gamma, beta):
    idx = item_input.astype(jnp.int32)
    b2 = b.reshape(1, HID_DIM)
    g2 = gamma.reshape(1, HID_DIM)
    be2 = beta.reshape(1, HID_DIM)

    x0 = _make_gather(0)(idx, table)
    x1 = _make_gather(1)(idx, table)

    w_spec = pl.BlockSpec((EMB_DIM, HID_DIM), lambda i: (0, 0))
    v_spec = pl.BlockSpec((1, HID_DIM), lambda i: (0, 0))
    x_spec = pl.BlockSpec((BB, EMB_DIM), lambda i: (i, 0))

    out_a = pl.pallas_call(
        _fc_first_body,
        grid=(BLOCKS_PER_CHUNK,),
        in_specs=[x_spec, w_spec, v_spec, v_spec, v_spec],
        out_specs=pl.BlockSpec((BB, HID_DIM), lambda i: (i, 0)),
        out_shape=jax.ShapeDtypeStruct((BATCH, HID_DIM), jnp.float32),
    )(x0, W, b2, g2, be2)

    out = pl.pallas_call(
        _fc_second_body,
        grid=(BLOCKS_PER_CHUNK,),
        in_specs=[x_spec, w_spec, v_spec, v_spec, v_spec,
                  pl.BlockSpec(memory_space=pl.ANY)],
        out_specs=pl.BlockSpec(
            (BB, HID_DIM), lambda i: (i + BLOCKS_PER_CHUNK, 0)),
        out_shape=jax.ShapeDtypeStruct((BATCH, HID_DIM), jnp.float32),
        input_output_aliases={5: 0},
    )(x1, W, b2, g2, be2, out_a)
    return out
